# baseline (speedup 1.0000x reference)
import functools

import jax
import jax.numpy as jnp
from jax import lax
from jax.experimental import pallas as pl
from jax.experimental.pallas import tpu as pltpu
from jax.experimental.pallas import tpu_sc as plsc

EMBED_DIM = 64
NUM_CORES = 2
NUM_SUBCORES = 16
NUM_WORKERS = NUM_CORES * NUM_SUBCORES


def _make_gather(total_rows: int, chunk: int):
    rows_per_w = total_rows // NUM_WORKERS
    n_chunks = rows_per_w // chunk
    mesh = plsc.VectorSubcoreMesh(core_axis_name="c", subcore_axis_name="s")

    @functools.partial(
        pl.kernel,
        mesh=mesh,
        out_type=jax.ShapeDtypeStruct((total_rows, EMBED_DIM), jnp.float32),
        scratch_types=[
            pltpu.VMEM((chunk,), jnp.int32),
            pltpu.VMEM((chunk, EMBED_DIM), jnp.float32),
            pltpu.SemaphoreType.DMA,
        ],
        compiler_params=pltpu.CompilerParams(use_tc_tiling_on_sc=False),
    )
    def gather(table_hbm, idx_hbm, out_hbm, idx_v, rows_v, sem):
        wid = lax.axis_index("s") * NUM_CORES + lax.axis_index("c")
        base = wid * rows_per_w

        def body(i, carry):
            off = base + i * chunk
            pltpu.sync_copy(idx_hbm.at[pl.ds(off, chunk)], idx_v)
            pltpu.async_copy(table_hbm.at[idx_v], rows_v, sem).wait()
            pltpu.sync_copy(rows_v, out_hbm.at[pl.ds(off, chunk)])
            return carry

        lax.fori_loop(0, n_chunks, body, 0)

    return gather


def kernel(IX, weight):
    b, t = IX.shape
    idx = IX.reshape(-1).astype(jnp.int32)
    out = _make_gather(b * t, 800)(weight, idx)
    return out.reshape(b, t, EMBED_DIM)


# speedup vs baseline: 1.0070x; 1.0070x over previous
import functools

import jax
import jax.numpy as jnp
from jax import lax
from jax.experimental import pallas as pl
from jax.experimental.pallas import tpu as pltpu
from jax.experimental.pallas import tpu_sc as plsc

EMBED_DIM = 64
NUM_CORES = 2
NUM_SUBCORES = 16
NUM_WORKERS = NUM_CORES * NUM_SUBCORES
CHUNK = 800
N_CHUNKS = 8


def _make_gather(total_rows: int):
    rows_per_w = total_rows // NUM_WORKERS
    assert rows_per_w == CHUNK * N_CHUNKS
    mesh = plsc.VectorSubcoreMesh(core_axis_name="c", subcore_axis_name="s")

    @functools.partial(
        pl.kernel,
        mesh=mesh,
        out_type=jax.ShapeDtypeStruct((total_rows, EMBED_DIM), jnp.float32),
        scratch_types=[
            pltpu.VMEM((N_CHUNKS, CHUNK), jnp.int32),
            pltpu.VMEM((CHUNK, EMBED_DIM), jnp.float32),
            pltpu.VMEM((CHUNK, EMBED_DIM), jnp.float32),
            pltpu.SemaphoreType.DMA,
            pltpu.SemaphoreType.DMA,
            pltpu.SemaphoreType.DMA,
            pltpu.SemaphoreType.DMA,
        ],
        compiler_params=pltpu.CompilerParams(use_tc_tiling_on_sc=False),
    )
    def gather(table_hbm, idx_hbm, out_hbm, idx_v, rows0, rows1, g0, g1, o0, o1):
        wid = lax.axis_index("s") * NUM_CORES + lax.axis_index("c")
        base = wid * rows_per_w
        pltpu.sync_copy(idx_hbm.at[wid], idx_v)

        rows = (rows0, rows1)
        gsem = (g0, g1)
        osem = (o0, o1)

        def start_gather(i):
            return pltpu.async_copy(table_hbm.at[idx_v.at[i]], rows[i % 2], gsem[i % 2])

        def start_out(i):
            return pltpu.async_copy(
                rows[i % 2], out_hbm.at[pl.ds(base + i * CHUNK, CHUNK)], osem[i % 2]
            )

        g = [None] * N_CHUNKS
        o = [None] * N_CHUNKS
        g[0] = start_gather(0)
        g[1] = start_gather(1)
        for i in range(N_CHUNKS):
            g[i].wait()
            o[i] = start_out(i)
            if i + 2 < N_CHUNKS:
                o[i].wait()
                g[i + 2] = start_gather(i + 2)
        o[N_CHUNKS - 2].wait()
        o[N_CHUNKS - 1].wait()

    return gather


def kernel(IX, weight):
    b, t = IX.shape
    total = b * t
    idx = IX.reshape(NUM_WORKERS, N_CHUNKS, CHUNK).astype(jnp.int32)
    out = _make_gather(total)(weight, idx)
    return out.reshape(b, t, EMBED_DIM)


# speedup vs baseline: 1.0146x; 1.0076x over previous
import functools

import jax
import jax.numpy as jnp
from jax import lax
from jax.experimental import pallas as pl
from jax.experimental.pallas import tpu as pltpu
from jax.experimental.pallas import tpu_sc as plsc

PAD_DIM = 128
EMBED_DIM = 64
NUM_CORES = 2
NUM_SUBCORES = 16
NUM_WORKERS = NUM_CORES * NUM_SUBCORES
CHUNK = 400
N_CHUNKS = 16


def _make_gather(total_rows: int):
    rows_per_w = total_rows // NUM_WORKERS
    assert rows_per_w == CHUNK * N_CHUNKS
    mesh = plsc.VectorSubcoreMesh(core_axis_name="c", subcore_axis_name="s")

    @functools.partial(
        pl.kernel,
        mesh=mesh,
        out_type=jax.ShapeDtypeStruct((total_rows, PAD_DIM), jnp.float32),
        scratch_types=[
            pltpu.VMEM((rows_per_w,), jnp.int32),
            pltpu.VMEM((CHUNK, PAD_DIM), jnp.float32),
            pltpu.VMEM((CHUNK, PAD_DIM), jnp.float32),
            pltpu.SemaphoreType.DMA,
            pltpu.SemaphoreType.DMA,
            pltpu.SemaphoreType.DMA,
            pltpu.SemaphoreType.DMA,
        ],
    )
    def gather(table_hbm, idx_hbm, out_hbm, idx_v, rows0, rows1, g0, g1, o0, o1):
        wid = lax.axis_index("s") * NUM_CORES + lax.axis_index("c")
        base = wid * rows_per_w
        pltpu.sync_copy(idx_hbm.at[pl.ds(base, rows_per_w)], idx_v)

        rows = (rows0, rows1)
        gsem = (g0, g1)
        osem = (o0, o1)

        def start_gather(i):
            return pltpu.async_copy(
                table_hbm.at[idx_v.at[pl.ds(i * CHUNK, CHUNK)]],
                rows[i % 2],
                gsem[i % 2],
            )

        def start_out(i):
            return pltpu.async_copy(
                rows[i % 2], out_hbm.at[pl.ds(base + i * CHUNK, CHUNK)], osem[i % 2]
            )

        g = [None] * N_CHUNKS
        o = [None] * N_CHUNKS
        g[0] = start_gather(0)
        g[1] = start_gather(1)
        for i in range(N_CHUNKS):
            g[i].wait()
            o[i] = start_out(i)
            if i + 2 < N_CHUNKS:
                o[i].wait()
                g[i + 2] = start_gather(i + 2)
        o[N_CHUNKS - 2].wait()
        o[N_CHUNKS - 1].wait()

    return gather


def kernel(IX, weight):
    b, t = IX.shape
    total = b * t
    wp = jnp.pad(weight, ((0, 0), (0, PAD_DIM - EMBED_DIM)))
    idx = IX.reshape(-1).astype(jnp.int32)
    out = _make_gather(total)(wp, idx)
    return out[:, :EMBED_DIM].reshape(b, t, EMBED_DIM)


# speedup vs baseline: 1.3453x; 1.3260x over previous
import functools

import jax
import jax.numpy as jnp
from jax import lax
from jax.experimental import pallas as pl
from jax.experimental.pallas import tpu as pltpu
from jax.experimental.pallas import tpu_sc as plsc

PAD_DIM = 128
EMBED_DIM = 64
NUM_CORES = 2
NUM_SUBCORES = 16
NUM_WORKERS = NUM_CORES * NUM_SUBCORES
CHUNK = 400
N_CHUNKS = 16

TBLOCK = 4096


def _transpose_block(in_ref, out_ref):
    x = in_ref[...]
    y = x.T
    out_ref[...] = jnp.concatenate(
        [y, jnp.zeros((TBLOCK, PAD_DIM - EMBED_DIM), jnp.float32)], axis=1
    )


def _relayout_table(wt):
    v = wt.shape[1]
    grid = (v + TBLOCK - 1) // TBLOCK
    return pl.pallas_call(
        _transpose_block,
        grid=(grid,),
        in_specs=[pl.BlockSpec((EMBED_DIM, TBLOCK), lambda n: (0, n))],
        out_specs=pl.BlockSpec((TBLOCK, PAD_DIM), lambda n: (n, 0)),
        out_shape=jax.ShapeDtypeStruct((v, PAD_DIM), jnp.float32),
    )(wt)


def _make_gather(total_rows: int):
    rows_per_w = total_rows // NUM_WORKERS
    assert rows_per_w == CHUNK * N_CHUNKS
    mesh = plsc.VectorSubcoreMesh(core_axis_name="c", subcore_axis_name="s")

    @functools.partial(
        pl.kernel,
        mesh=mesh,
        out_type=jax.ShapeDtypeStruct((total_rows, PAD_DIM), jnp.float32),
        scratch_types=[
            pltpu.VMEM((rows_per_w,), jnp.int32),
            pltpu.VMEM((CHUNK, PAD_DIM), jnp.float32),
            pltpu.VMEM((CHUNK, PAD_DIM), jnp.float32),
            pltpu.SemaphoreType.DMA,
            pltpu.SemaphoreType.DMA,
            pltpu.SemaphoreType.DMA,
            pltpu.SemaphoreType.DMA,
        ],
    )
    def gather(table_hbm, idx_hbm, out_hbm, idx_v, rows0, rows1, g0, g1, o0, o1):
        wid = lax.axis_index("s") * NUM_CORES + lax.axis_index("c")
        base = wid * rows_per_w
        pltpu.sync_copy(idx_hbm.at[pl.ds(base, rows_per_w)], idx_v)

        rows = (rows0, rows1)
        gsem = (g0, g1)
        osem = (o0, o1)

        def start_gather(i):
            return pltpu.async_copy(
                table_hbm.at[idx_v.at[pl.ds(i * CHUNK, CHUNK)]],
                rows[i % 2],
                gsem[i % 2],
            )

        def start_out(i):
            return pltpu.async_copy(
                rows[i % 2], out_hbm.at[pl.ds(base + i * CHUNK, CHUNK)], osem[i % 2]
            )

        g = [None] * N_CHUNKS
        o = [None] * N_CHUNKS
        g[0] = start_gather(0)
        g[1] = start_gather(1)
        for i in range(N_CHUNKS):
            g[i].wait()
            o[i] = start_out(i)
            if i + 2 < N_CHUNKS:
                o[i].wait()
                g[i + 2] = start_gather(i + 2)
        o[N_CHUNKS - 2].wait()
        o[N_CHUNKS - 1].wait()

    return gather


def kernel(IX, weight):
    b, t = IX.shape
    total = b * t
    wp = _relayout_table(weight.T)
    idx = IX.reshape(-1).astype(jnp.int32)
    out = _make_gather(total)(wp, idx)
    return out[:, :EMBED_DIM].reshape(b, t, EMBED_DIM)
